# Initial kernel scaffold; baseline (speedup 1.0000x reference)
import functools

import jax
import jax.numpy as jnp
from jax import lax
from jax.experimental import pallas as pl
from jax.experimental.pallas import tpu as pltpu
from jax.experimental.pallas import tpu_sc as plsc

NC = 2
NS = 16
NW = NC * NS
LANES = 16
CHUNK = 128
HW = 16


def _fill_2d(ref, rows, width, value):
    vals = jnp.full((LANES,), value, dtype=jnp.float32)

    @pl.loop(0, rows)
    def _(i):
        for j in range(width // LANES):
            ref[i, pl.ds(j * LANES, LANES)] = vals


def _copy_rows(src_ref, dst_ref, dst_base, n_rows, buf_rows):
    full, rem = divmod(n_rows, buf_rows)
    for t in range(full):
        pltpu.sync_copy(
            src_ref,
            dst_ref.at[pl.ds(dst_base + t * buf_rows, buf_rows)])
    if rem:
        pltpu.sync_copy(src_ref.at[pl.ds(0, rem)],
                        dst_ref.at[pl.ds(dst_base + full * buf_rows, rem)])


def _hist_sc(dst_i32, n_nodes):
    n_edges = dst_i32.shape[0]
    n_chunks = n_edges // CHUNK
    iters = pl.cdiv(n_chunks, NW)
    rows_per_sub = n_nodes // NS
    mesh = plsc.VectorSubcoreMesh(core_axis_name="c", subcore_axis_name="s")

    @functools.partial(
        pl.kernel, mesh=mesh,
        out_type=jax.ShapeDtypeStruct((NC, n_nodes, HW), jnp.float32),
        scratch_types=[
            pltpu.VMEM_SHARED((n_nodes, HW), jnp.float32),
            pltpu.VMEM((CHUNK,), jnp.int32),
            pltpu.VMEM((CHUNK, HW), jnp.float32),
        ],
    )
    def hist_kernel(dst_hbm, hist_hbm, deg_sp, idx_v, val_v):
        c = lax.axis_index("c")
        s = lax.axis_index("s")
        w = s * NC + c

        _fill_2d(val_v, CHUNK, HW, 0.0)
        _copy_rows(val_v, deg_sp, s * rows_per_sub, rows_per_sub, CHUNK)
        plsc.subcore_barrier()

        _fill_2d(val_v, CHUNK, HW, 1.0)

        @pl.loop(0, iters)
        def _(i):
            chunk = i * NW + w

            @pl.when(chunk < n_chunks)
            def _():
                pltpu.sync_copy(dst_hbm.at[pl.ds(chunk * CHUNK, CHUNK)], idx_v)
                pltpu.sync_copy(val_v, deg_sp.at[idx_v], add=True)

        plsc.subcore_barrier()
        pltpu.sync_copy(deg_sp.at[pl.ds(s * rows_per_sub, rows_per_sub)],
                        hist_hbm.at[c, pl.ds(s * rows_per_sub, rows_per_sub)])

    return hist_kernel(dst_i32)


def _scatter_sc(g, src_i32, dst_i32):
    n_nodes, d = g.shape
    n_edges = src_i32.shape[0]
    n_chunks = n_edges // CHUNK
    iters = pl.cdiv(n_chunks, NW)
    rows_per_sub = n_nodes // NS
    mesh = plsc.VectorSubcoreMesh(core_axis_name="c", subcore_axis_name="s")

    @functools.partial(
        pl.kernel, mesh=mesh,
        out_type=jax.ShapeDtypeStruct((NC, n_nodes, d), jnp.float32),
        scratch_types=[
            pltpu.VMEM_SHARED((n_nodes, d), jnp.float32),
            pltpu.VMEM((CHUNK,), jnp.int32),
            pltpu.VMEM((CHUNK,), jnp.int32),
            pltpu.VMEM((CHUNK, d), jnp.float32),
            pltpu.SemaphoreType.DMA,
        ],
    )
    def scatter_kernel(g_hbm, src_hbm, dst_hbm, out_hbm,
                       acc_sp, src_v, dst_v, rows_v, sem):
        c = lax.axis_index("c")
        s = lax.axis_index("s")
        w = s * NC + c

        _fill_2d(rows_v, CHUNK, d, 0.0)
        _copy_rows(rows_v, acc_sp, s * rows_per_sub, rows_per_sub, CHUNK)
        plsc.subcore_barrier()

        @pl.loop(0, iters)
        def _(i):
            chunk = i * NW + w

            @pl.when(chunk < n_chunks)
            def _():
                pltpu.sync_copy(src_hbm.at[pl.ds(chunk * CHUNK, CHUNK)], src_v)
                gather = pltpu.async_copy(g_hbm.at[src_v], rows_v, sem)
                pltpu.sync_copy(dst_hbm.at[pl.ds(chunk * CHUNK, CHUNK)], dst_v)
                gather.wait()
                pltpu.sync_copy(rows_v, acc_sp.at[dst_v], add=True)

        plsc.subcore_barrier()
        pltpu.sync_copy(acc_sp.at[pl.ds(s * rows_per_sub, rows_per_sub)],
                        out_hbm.at[c, pl.ds(s * rows_per_sub, rows_per_sub)])

    return scatter_kernel(g, src_i32, dst_i32)


def _matmul_tc(x, W):
    n, d_in = x.shape
    d_out = W.shape[1]
    bm = 2000

    def body(x_ref, w_ref, h_ref):
        h_ref[...] = lax.dot_general(
            x_ref[...], w_ref[...], (((1,), (0,)), ((), ())),
            precision=lax.Precision.HIGHEST,
            preferred_element_type=jnp.float32)

    return pl.pallas_call(
        body,
        grid=(n // bm,),
        in_specs=[
            pl.BlockSpec((bm, d_in), lambda i: (i, 0)),
            pl.BlockSpec((d_in, d_out), lambda i: (0, 0)),
        ],
        out_specs=pl.BlockSpec((bm, d_out), lambda i: (i, 0)),
        out_shape=jax.ShapeDtypeStruct((n, d_out), jnp.float32),
    )(x, W)


def _scale_tc(h, hist):
    n, d = h.shape
    bm = 2000

    def body(h_ref, hist_ref, g_ref, dis_ref):
        deg = hist_ref[0, :, 0:1] + hist_ref[1, :, 0:1] + 1.0
        dis = lax.rsqrt(deg)
        dis_ref[...] = dis
        g_ref[...] = h_ref[...] * dis

    return pl.pallas_call(
        body,
        grid=(n // bm,),
        in_specs=[
            pl.BlockSpec((bm, d), lambda i: (i, 0)),
            pl.BlockSpec((NC, bm, HW), lambda i: (0, i, 0)),
        ],
        out_specs=[
            pl.BlockSpec((bm, d), lambda i: (i, 0)),
            pl.BlockSpec((bm, 1), lambda i: (i, 0)),
        ],
        out_shape=[
            jax.ShapeDtypeStruct((n, d), jnp.float32),
            jax.ShapeDtypeStruct((n, 1), jnp.float32),
        ],
    )(h, hist)


def _combine_tc(partials, g, dis, b2):
    n, d = g.shape
    bm = 2000

    def body(p_ref, g_ref, dis_ref, b_ref, o_ref):
        o_ref[...] = (dis_ref[...] * (p_ref[0] + p_ref[1] + g_ref[...])
                      + b_ref[...])

    return pl.pallas_call(
        body,
        grid=(n // bm,),
        in_specs=[
            pl.BlockSpec((NC, bm, d), lambda i: (0, i, 0)),
            pl.BlockSpec((bm, d), lambda i: (i, 0)),
            pl.BlockSpec((bm, 1), lambda i: (i, 0)),
            pl.BlockSpec((1, d), lambda i: (0, 0)),
        ],
        out_specs=pl.BlockSpec((bm, d), lambda i: (i, 0)),
        out_shape=jax.ShapeDtypeStruct((n, d), jnp.float32),
    )(partials, g, dis, b2)


def kernel(x, edge_index, W, b):
    ei = edge_index.astype(jnp.int32)
    src, dst = ei[0], ei[1]

    h = _matmul_tc(x, W)
    hist = _hist_sc(dst, x.shape[0])
    g, dis = _scale_tc(h, hist)
    partials = _scatter_sc(g, src, dst)
    return _combine_tc(partials, g, dis, b.reshape(1, -1))


# speedup vs baseline: 24.2496x; 24.2496x over previous
import functools

import jax
import jax.numpy as jnp
from jax import lax
from jax.experimental import pallas as pl
from jax.experimental.pallas import tpu as pltpu
from jax.experimental.pallas import tpu_sc as plsc

NC = 2
NS = 16
NW = NC * NS
LANES = 16
CHUNK = 128
HW = 16


def _fill_2d(ref, rows, width, value):
    vals = jnp.full((LANES,), value, dtype=jnp.float32)

    @pl.loop(0, rows)
    def _(i):
        for j in range(width // LANES):
            ref[i, pl.ds(j * LANES, LANES)] = vals


def _copy_rows(src_ref, dst_ref, dst_base, n_rows, buf_rows):
    full, rem = divmod(n_rows, buf_rows)
    for t in range(full):
        pltpu.sync_copy(
            src_ref,
            dst_ref.at[pl.ds(dst_base + t * buf_rows, buf_rows)])
    if rem:
        pltpu.sync_copy(src_ref.at[pl.ds(0, rem)],
                        dst_ref.at[pl.ds(dst_base + full * buf_rows, rem)])


def _hist_sc(dst_i32, n_nodes):
    n_edges = dst_i32.shape[0]
    n_chunks = n_edges // CHUNK
    iters = pl.cdiv(n_chunks, NW)
    n_pad = ((n_nodes + NS * 8 - 1) // (NS * 8)) * (NS * 8)
    rows_per_sub = n_pad // NS
    mesh = plsc.VectorSubcoreMesh(core_axis_name="c", subcore_axis_name="s")

    @functools.partial(
        pl.kernel, mesh=mesh,
        out_type=jax.ShapeDtypeStruct((NC, n_pad, HW), jnp.float32),
        scratch_types=[
            pltpu.VMEM_SHARED((n_pad, HW), jnp.float32),
            pltpu.VMEM((CHUNK,), jnp.int32),
            pltpu.VMEM((CHUNK, HW), jnp.float32),
        ],
    )
    def hist_kernel(dst_hbm, hist_hbm, deg_sp, idx_v, val_v):
        c = lax.axis_index("c")
        s = lax.axis_index("s")
        w = s * NC + c

        _fill_2d(val_v, CHUNK, HW, 0.0)
        _copy_rows(val_v, deg_sp, s * rows_per_sub, rows_per_sub, CHUNK)
        plsc.subcore_barrier()

        _fill_2d(val_v, CHUNK, HW, 1.0)

        @pl.loop(0, iters)
        def _(i):
            chunk = i * NW + w

            @pl.when(chunk < n_chunks)
            def _():
                pltpu.sync_copy(dst_hbm.at[pl.ds(chunk * CHUNK, CHUNK)], idx_v)
                pltpu.sync_copy(val_v, deg_sp.at[idx_v], add=True)

        plsc.subcore_barrier()
        pltpu.sync_copy(deg_sp.at[pl.ds(s * rows_per_sub, rows_per_sub)],
                        hist_hbm.at[c, pl.ds(s * rows_per_sub, rows_per_sub)])

    return hist_kernel(dst_i32)


def _scatter_sc(g, src_i32, dst_i32):
    n_nodes, d = g.shape
    n_edges = src_i32.shape[0]
    n_chunks = n_edges // CHUNK
    iters = pl.cdiv(n_chunks, NW)
    n_pad = ((n_nodes + NS * 8 - 1) // (NS * 8)) * (NS * 8)
    rows_per_sub = n_pad // NS
    mesh = plsc.VectorSubcoreMesh(core_axis_name="c", subcore_axis_name="s")

    @functools.partial(
        pl.kernel, mesh=mesh,
        out_type=jax.ShapeDtypeStruct((NC, n_pad, d), jnp.float32),
        scratch_types=[
            pltpu.VMEM_SHARED((n_pad, d), jnp.float32),
            pltpu.VMEM((CHUNK,), jnp.int32),
            pltpu.VMEM((CHUNK,), jnp.int32),
            pltpu.VMEM((CHUNK, d), jnp.float32),
            pltpu.SemaphoreType.DMA,
        ],
    )
    def scatter_kernel(g_hbm, src_hbm, dst_hbm, out_hbm,
                       acc_sp, src_v, dst_v, rows_v, sem):
        c = lax.axis_index("c")
        s = lax.axis_index("s")
        w = s * NC + c

        _fill_2d(rows_v, CHUNK, d, 0.0)
        _copy_rows(rows_v, acc_sp, s * rows_per_sub, rows_per_sub, CHUNK)
        plsc.subcore_barrier()

        @pl.loop(0, iters)
        def _(i):
            chunk = i * NW + w

            @pl.when(chunk < n_chunks)
            def _():
                pltpu.sync_copy(src_hbm.at[pl.ds(chunk * CHUNK, CHUNK)], src_v)
                gather = pltpu.async_copy(g_hbm.at[src_v], rows_v, sem)
                pltpu.sync_copy(dst_hbm.at[pl.ds(chunk * CHUNK, CHUNK)], dst_v)
                gather.wait()
                pltpu.sync_copy(rows_v, acc_sp.at[dst_v], add=True)

        plsc.subcore_barrier()
        pltpu.sync_copy(acc_sp.at[pl.ds(s * rows_per_sub, rows_per_sub)],
                        out_hbm.at[c, pl.ds(s * rows_per_sub, rows_per_sub)])

    return scatter_kernel(g, src_i32, dst_i32)


def _matmul_tc(x, W):
    n, d_in = x.shape
    d_out = W.shape[1]
    bm = 2000

    def body(x_ref, w_ref, h_ref):
        h_ref[...] = lax.dot_general(
            x_ref[...], w_ref[...], (((1,), (0,)), ((), ())),
            precision=lax.Precision.HIGHEST,
            preferred_element_type=jnp.float32)

    return pl.pallas_call(
        body,
        grid=(n // bm,),
        in_specs=[
            pl.BlockSpec((bm, d_in), lambda i: (i, 0)),
            pl.BlockSpec((d_in, d_out), lambda i: (0, 0)),
        ],
        out_specs=pl.BlockSpec((bm, d_out), lambda i: (i, 0)),
        out_shape=jax.ShapeDtypeStruct((n, d_out), jnp.float32),
    )(x, W)


def _scale_tc(h, hist):
    n, d = h.shape
    bm = 2000

    def body(h_ref, hist_ref, g_ref, dis_ref):
        deg = hist_ref[0, :, 0:1] + hist_ref[1, :, 0:1] + 1.0
        dis = lax.rsqrt(deg)
        dis_ref[...] = dis
        g_ref[...] = h_ref[...] * dis

    return pl.pallas_call(
        body,
        grid=(n // bm,),
        in_specs=[
            pl.BlockSpec((bm, d), lambda i: (i, 0)),
            pl.BlockSpec((NC, bm, HW), lambda i: (0, i, 0)),
        ],
        out_specs=[
            pl.BlockSpec((bm, d), lambda i: (i, 0)),
            pl.BlockSpec((bm, 1), lambda i: (i, 0)),
        ],
        out_shape=[
            jax.ShapeDtypeStruct((n, d), jnp.float32),
            jax.ShapeDtypeStruct((n, 1), jnp.float32),
        ],
    )(h, hist)


def _combine_tc(partials, g, dis, b2):
    n, d = g.shape
    bm = 2000

    def body(p_ref, g_ref, dis_ref, b_ref, o_ref):
        o_ref[...] = (dis_ref[...] * (p_ref[0] + p_ref[1] + g_ref[...])
                      + b_ref[...])

    return pl.pallas_call(
        body,
        grid=(n // bm,),
        in_specs=[
            pl.BlockSpec((NC, bm, d), lambda i: (0, i, 0)),
            pl.BlockSpec((bm, d), lambda i: (i, 0)),
            pl.BlockSpec((bm, 1), lambda i: (i, 0)),
            pl.BlockSpec((1, d), lambda i: (0, 0)),
        ],
        out_specs=pl.BlockSpec((bm, d), lambda i: (i, 0)),
        out_shape=jax.ShapeDtypeStruct((n, d), jnp.float32),
    )(partials, g, dis, b2)


def kernel(x, edge_index, W, b):
    ei = edge_index.astype(jnp.int32)
    src, dst = ei[0], ei[1]

    h = _matmul_tc(x, W)
    hist = _hist_sc(dst, x.shape[0])
    g, dis = _scale_tc(h, hist)
    partials = _scatter_sc(g, src, dst)
    return _combine_tc(partials, g, dis, b.reshape(1, -1))


# speedup vs baseline: 33.6687x; 1.3884x over previous
import functools

import jax
import jax.numpy as jnp
from jax import lax
from jax.experimental import pallas as pl
from jax.experimental.pallas import tpu as pltpu
from jax.experimental.pallas import tpu_sc as plsc

NC = 2
NS = 16
NW = NC * NS
LANES = 16
CHUNK = 128
HW = 16
NBUF = 4
RBUF = 2


def _fill_2d(ref, rows, width, value):
    vals = jnp.full((LANES,), value, dtype=jnp.float32)

    @pl.loop(0, rows)
    def _(i):
        for j in range(width // LANES):
            ref[i, pl.ds(j * LANES, LANES)] = vals


def _copy_rows(src_ref, dst_ref, dst_base, n_rows, buf_rows):
    full, rem = divmod(n_rows, buf_rows)
    for t in range(full):
        pltpu.sync_copy(
            src_ref,
            dst_ref.at[pl.ds(dst_base + t * buf_rows, buf_rows)])
    if rem:
        pltpu.sync_copy(src_ref.at[pl.ds(0, rem)],
                        dst_ref.at[pl.ds(dst_base + full * buf_rows, rem)])


def _pad_chunks(idx_1d):
    n_chunks = idx_1d.shape[0] // CHUNK
    per_w = ((n_chunks + NW - 1) // NW + 7) // 8 * 8
    rows = idx_1d.reshape(n_chunks, CHUNK)
    return jnp.pad(rows, ((0, per_w * NW - n_chunks), (0, 0))), n_chunks, per_w


def _hist_sc(dst2, n_chunks, per_w, n_nodes):
    n_pad = ((n_nodes + NS * 8 - 1) // (NS * 8)) * (NS * 8)
    rows_per_sub = n_pad // NS
    outer = per_w // NBUF
    mesh = plsc.VectorSubcoreMesh(core_axis_name="c", subcore_axis_name="s")

    @functools.partial(
        pl.kernel, mesh=mesh,
        out_type=jax.ShapeDtypeStruct((NC, n_pad, HW), jnp.float32),
        scratch_types=[
            pltpu.VMEM_SHARED((n_pad, HW), jnp.float32),
            pltpu.VMEM((per_w, CHUNK), jnp.int32),
            pltpu.VMEM((CHUNK, HW), jnp.float32),
        ] + [pltpu.SemaphoreType.DMA] * NBUF,
    )
    def hist_kernel(dst_hbm, hist_hbm, deg_sp, idx_v, val_v, *sems):
        c = lax.axis_index("c")
        s = lax.axis_index("s")
        w = s * NC + c
        base = w * per_w

        _fill_2d(val_v, CHUNK, HW, 0.0)
        _copy_rows(val_v, deg_sp, s * rows_per_sub, rows_per_sub, CHUNK)
        pltpu.sync_copy(dst_hbm.at[pl.ds(base, per_w)], idx_v)
        _fill_2d(val_v, CHUNK, HW, 1.0)
        plsc.subcore_barrier()

        @pl.loop(0, outer)
        def _(o):
            for b in range(NBUF):
                i = o * NBUF + b

                @pl.when(base + i < n_chunks)
                def _():
                    @pl.when(o > 0)
                    def _():
                        pltpu.make_async_copy(
                            val_v, deg_sp.at[idx_v.at[i - NBUF]],
                            sems[b]).wait()
                    pltpu.async_copy(val_v, deg_sp.at[idx_v.at[i]],
                                     sems[b], add=True)

        for b in range(NBUF):
            last = (outer - 1) * NBUF + b

            @pl.when(base + b < n_chunks)
            def _():
                i_fin = jnp.minimum(last, n_chunks - 1 - base)
                pltpu.make_async_copy(val_v, deg_sp.at[idx_v.at[i_fin]],
                                      sems[b]).wait()

        plsc.subcore_barrier()
        pltpu.sync_copy(deg_sp.at[pl.ds(s * rows_per_sub, rows_per_sub)],
                        hist_hbm.at[c, pl.ds(s * rows_per_sub, rows_per_sub)])

    return hist_kernel(dst2)


def _scatter_sc(g, src_flat, dst_flat):
    n_nodes, d = g.shape
    n_edges = src_flat.shape[0]
    n_chunks = n_edges // CHUNK
    outer = (pl.cdiv(n_chunks, NW) + RBUF - 1) // RBUF
    n_pad = ((n_nodes + NS * 8 - 1) // (NS * 8)) * (NS * 8)
    rows_per_sub = n_pad // NS
    mesh = plsc.VectorSubcoreMesh(core_axis_name="c", subcore_axis_name="s")

    @functools.partial(
        pl.kernel, mesh=mesh,
        out_type=jax.ShapeDtypeStruct((NC, n_pad, d), jnp.float32),
        scratch_types=[pltpu.VMEM_SHARED((n_pad, d), jnp.float32)]
        + [pltpu.VMEM((CHUNK,), jnp.int32)] * (2 * RBUF)
        + [pltpu.VMEM((CHUNK, d), jnp.float32)] * RBUF
        + [pltpu.SemaphoreType.DMA] * (4 * RBUF),
    )
    def scatter_kernel(g_hbm, src_hbm, dst_hbm, out_hbm, acc_sp, *scr):
        srcv = scr[:RBUF]
        dstv = scr[RBUF:2 * RBUF]
        rows = scr[2 * RBUF:3 * RBUF]
        isem = scr[3 * RBUF:4 * RBUF]
        jsem = scr[4 * RBUF:5 * RBUF]
        gsem = scr[5 * RBUF:6 * RBUF]
        ssem = scr[6 * RBUF:7 * RBUF]
        c = lax.axis_index("c")
        s = lax.axis_index("s")
        w = s * NC + c

        _fill_2d(rows[0], CHUNK, d, 0.0)
        _copy_rows(rows[0], acc_sp, s * rows_per_sub, rows_per_sub, CHUNK)
        plsc.subcore_barrier()

        @pl.loop(0, outer)
        def _(o):
            for b in range(RBUF):
                chunk = (o * RBUF + b) * NW + w

                @pl.when(chunk < n_chunks)
                def _():
                    @pl.when(o > 0)
                    def _():
                        pltpu.make_async_copy(
                            rows[b], acc_sp.at[dstv[b]], ssem[b]).wait()
                    pltpu.async_copy(
                        src_hbm.at[pl.ds(chunk * CHUNK, CHUNK)],
                        srcv[b], isem[b])
                    pltpu.async_copy(
                        dst_hbm.at[pl.ds(chunk * CHUNK, CHUNK)],
                        dstv[b], jsem[b])
            for b in range(RBUF):
                chunk = (o * RBUF + b) * NW + w

                @pl.when(chunk < n_chunks)
                def _():
                    pltpu.make_async_copy(
                        src_hbm.at[pl.ds(chunk * CHUNK, CHUNK)],
                        srcv[b], isem[b]).wait()
                    pltpu.async_copy(g_hbm.at[srcv[b]], rows[b], gsem[b])
            for b in range(RBUF):
                chunk = (o * RBUF + b) * NW + w

                @pl.when(chunk < n_chunks)
                def _():
                    pltpu.make_async_copy(
                        g_hbm.at[srcv[b]], rows[b], gsem[b]).wait()
                    pltpu.make_async_copy(
                        dst_hbm.at[pl.ds(chunk * CHUNK, CHUNK)],
                        dstv[b], jsem[b]).wait()
                    pltpu.async_copy(rows[b], acc_sp.at[dstv[b]],
                                     ssem[b], add=True)

        for b in range(RBUF):
            @pl.when(b * NW + w < n_chunks)
            def _():
                pltpu.make_async_copy(rows[b], acc_sp.at[dstv[b]],
                                      ssem[b]).wait()

        plsc.subcore_barrier()
        pltpu.sync_copy(acc_sp.at[pl.ds(s * rows_per_sub, rows_per_sub)],
                        out_hbm.at[c, pl.ds(s * rows_per_sub, rows_per_sub)])

    return scatter_kernel(g, src_flat, dst_flat)


def _matmul_tc(x, W):
    n, d_in = x.shape
    d_out = W.shape[1]
    bm = 2000

    def body(x_ref, w_ref, h_ref):
        h_ref[...] = lax.dot_general(
            x_ref[...], w_ref[...], (((1,), (0,)), ((), ())),
            precision=lax.Precision.HIGHEST,
            preferred_element_type=jnp.float32)

    return pl.pallas_call(
        body,
        grid=(n // bm,),
        in_specs=[
            pl.BlockSpec((bm, d_in), lambda i: (i, 0)),
            pl.BlockSpec((d_in, d_out), lambda i: (0, 0)),
        ],
        out_specs=pl.BlockSpec((bm, d_out), lambda i: (i, 0)),
        out_shape=jax.ShapeDtypeStruct((n, d_out), jnp.float32),
    )(x, W)


def _scale_tc(h, hist):
    n, d = h.shape
    bm = 2000

    def body(h_ref, hist_ref, g_ref, dis_ref):
        deg = hist_ref[0, :, 0:1] + hist_ref[1, :, 0:1] + 1.0
        dis = lax.rsqrt(deg)
        dis_ref[...] = dis
        g_ref[...] = h_ref[...] * dis

    return pl.pallas_call(
        body,
        grid=(n // bm,),
        in_specs=[
            pl.BlockSpec((bm, d), lambda i: (i, 0)),
            pl.BlockSpec((NC, bm, HW), lambda i: (0, i, 0)),
        ],
        out_specs=[
            pl.BlockSpec((bm, d), lambda i: (i, 0)),
            pl.BlockSpec((bm, 1), lambda i: (i, 0)),
        ],
        out_shape=[
            jax.ShapeDtypeStruct((n, d), jnp.float32),
            jax.ShapeDtypeStruct((n, 1), jnp.float32),
        ],
    )(h, hist)


def _combine_tc(partials, g, dis, b2):
    n, d = g.shape
    bm = 2000

    def body(p_ref, g_ref, dis_ref, b_ref, o_ref):
        o_ref[...] = (dis_ref[...] * (p_ref[0] + p_ref[1] + g_ref[...])
                      + b_ref[...])

    return pl.pallas_call(
        body,
        grid=(n // bm,),
        in_specs=[
            pl.BlockSpec((NC, bm, d), lambda i: (0, i, 0)),
            pl.BlockSpec((bm, d), lambda i: (i, 0)),
            pl.BlockSpec((bm, 1), lambda i: (i, 0)),
            pl.BlockSpec((1, d), lambda i: (0, 0)),
        ],
        out_specs=pl.BlockSpec((bm, d), lambda i: (i, 0)),
        out_shape=jax.ShapeDtypeStruct((n, d), jnp.float32),
    )(partials, g, dis, b2)


def kernel(x, edge_index, W, b):
    ei = edge_index.astype(jnp.int32)
    dst2, n_chunks, per_w = _pad_chunks(ei[1])

    h = _matmul_tc(x, W)
    hist = _hist_sc(dst2, n_chunks, per_w, x.shape[0])
    g, dis = _scale_tc(h, hist)
    partials = _scatter_sc(g, ei[0], ei[1])
    return _combine_tc(partials, g, dis, b.reshape(1, -1))


# speedup vs baseline: 37.0625x; 1.1008x over previous
import functools

import jax
import jax.numpy as jnp
from jax import lax
from jax.experimental import pallas as pl
from jax.experimental.pallas import tpu as pltpu
from jax.experimental.pallas import tpu_sc as plsc

NC = 2
NS = 16
NW = NC * NS
LANES = 16
CHUNK = 128
HW = 16
NBUF = 4
RBUF = 3


def _fill_2d(ref, rows, width, value):
    vals = jnp.full((LANES,), value, dtype=jnp.float32)

    @pl.loop(0, rows)
    def _(i):
        for j in range(width // LANES):
            ref[i, pl.ds(j * LANES, LANES)] = vals


def _copy_rows(src_ref, dst_ref, dst_base, n_rows, buf_rows):
    full, rem = divmod(n_rows, buf_rows)
    for t in range(full):
        pltpu.sync_copy(
            src_ref,
            dst_ref.at[pl.ds(dst_base + t * buf_rows, buf_rows)])
    if rem:
        pltpu.sync_copy(src_ref.at[pl.ds(0, rem)],
                        dst_ref.at[pl.ds(dst_base + full * buf_rows, rem)])


def _pad_chunks(idx_1d):
    n_chunks = idx_1d.shape[0] // CHUNK
    per_w = ((n_chunks + NW - 1) // NW + 7) // 8 * 8
    rows = idx_1d.reshape(n_chunks, CHUNK)
    return jnp.pad(rows, ((0, per_w * NW - n_chunks), (0, 0))), n_chunks, per_w


def _hist_sc(dst2, n_chunks, per_w, n_nodes):
    n_pad = ((n_nodes + NS * 8 - 1) // (NS * 8)) * (NS * 8)
    rows_per_sub = n_pad // NS
    outer = per_w // NBUF
    mesh = plsc.VectorSubcoreMesh(core_axis_name="c", subcore_axis_name="s")

    @functools.partial(
        pl.kernel, mesh=mesh,
        out_type=jax.ShapeDtypeStruct((NC, n_pad, HW), jnp.float32),
        scratch_types=[
            pltpu.VMEM_SHARED((n_pad, HW), jnp.float32),
            pltpu.VMEM((per_w, CHUNK), jnp.int32),
            pltpu.VMEM((CHUNK, HW), jnp.float32),
        ] + [pltpu.SemaphoreType.DMA] * NBUF,
    )
    def hist_kernel(dst_hbm, hist_hbm, deg_sp, idx_v, val_v, *sems):
        c = lax.axis_index("c")
        s = lax.axis_index("s")
        w = s * NC + c
        base = w * per_w

        _fill_2d(val_v, CHUNK, HW, 0.0)
        _copy_rows(val_v, deg_sp, s * rows_per_sub, rows_per_sub, CHUNK)
        pltpu.sync_copy(dst_hbm.at[pl.ds(base, per_w)], idx_v)
        _fill_2d(val_v, CHUNK, HW, 1.0)
        plsc.subcore_barrier()

        @pl.loop(0, outer)
        def _(o):
            for b in range(NBUF):
                i = o * NBUF + b

                @pl.when(base + i < n_chunks)
                def _():
                    @pl.when(o > 0)
                    def _():
                        pltpu.make_async_copy(
                            val_v, deg_sp.at[idx_v.at[i - NBUF]],
                            sems[b]).wait()
                    pltpu.async_copy(val_v, deg_sp.at[idx_v.at[i]],
                                     sems[b], add=True)

        for b in range(NBUF):
            last = (outer - 1) * NBUF + b

            @pl.when(base + b < n_chunks)
            def _():
                i_fin = jnp.minimum(last, n_chunks - 1 - base)
                pltpu.make_async_copy(val_v, deg_sp.at[idx_v.at[i_fin]],
                                      sems[b]).wait()

        plsc.subcore_barrier()
        pltpu.sync_copy(deg_sp.at[pl.ds(s * rows_per_sub, rows_per_sub)],
                        hist_hbm.at[c, pl.ds(s * rows_per_sub, rows_per_sub)])

    return hist_kernel(dst2)


def _scatter_sc(g, src_flat, dst_flat):
    n_nodes, d = g.shape
    n_edges = src_flat.shape[0]
    n_chunks = n_edges // CHUNK
    outer = (pl.cdiv(n_chunks, NW) + RBUF - 1) // RBUF
    n_pad = ((n_nodes + NS * 8 - 1) // (NS * 8)) * (NS * 8)
    rows_per_sub = n_pad // NS
    mesh = plsc.VectorSubcoreMesh(core_axis_name="c", subcore_axis_name="s")

    @functools.partial(
        pl.kernel, mesh=mesh,
        out_type=jax.ShapeDtypeStruct((NC, n_pad, d), jnp.float32),
        scratch_types=[pltpu.VMEM_SHARED((n_pad, d), jnp.float32)]
        + [pltpu.VMEM((CHUNK,), jnp.int32)] * (2 * RBUF)
        + [pltpu.VMEM((CHUNK, d), jnp.float32)] * RBUF
        + [pltpu.SemaphoreType.DMA] * (4 * RBUF),
    )
    def scatter_kernel(g_hbm, src_hbm, dst_hbm, out_hbm, acc_sp, *scr):
        srcv = scr[:RBUF]
        dstv = scr[RBUF:2 * RBUF]
        rows = scr[2 * RBUF:3 * RBUF]
        isem = scr[3 * RBUF:4 * RBUF]
        jsem = scr[4 * RBUF:5 * RBUF]
        gsem = scr[5 * RBUF:6 * RBUF]
        ssem = scr[6 * RBUF:7 * RBUF]
        c = lax.axis_index("c")
        s = lax.axis_index("s")
        w = s * NC + c

        _fill_2d(rows[0], CHUNK, d, 0.0)
        _copy_rows(rows[0], acc_sp, s * rows_per_sub, rows_per_sub, CHUNK)
        plsc.subcore_barrier()

        @pl.loop(0, outer)
        def _(o):
            for b in range(RBUF):
                chunk = (o * RBUF + b) * NW + w

                @pl.when(chunk < n_chunks)
                def _():
                    @pl.when(o > 0)
                    def _():
                        pltpu.make_async_copy(
                            rows[b], acc_sp.at[dstv[b]], ssem[b]).wait()
                    pltpu.async_copy(
                        src_hbm.at[pl.ds(chunk * CHUNK, CHUNK)],
                        srcv[b], isem[b])
                    pltpu.async_copy(
                        dst_hbm.at[pl.ds(chunk * CHUNK, CHUNK)],
                        dstv[b], jsem[b])
            for b in range(RBUF):
                chunk = (o * RBUF + b) * NW + w

                @pl.when(chunk < n_chunks)
                def _():
                    pltpu.make_async_copy(
                        src_hbm.at[pl.ds(chunk * CHUNK, CHUNK)],
                        srcv[b], isem[b]).wait()
                    pltpu.async_copy(g_hbm.at[srcv[b]], rows[b], gsem[b])
            for b in range(RBUF):
                chunk = (o * RBUF + b) * NW + w

                @pl.when(chunk < n_chunks)
                def _():
                    pltpu.make_async_copy(
                        g_hbm.at[srcv[b]], rows[b], gsem[b]).wait()
                    pltpu.make_async_copy(
                        dst_hbm.at[pl.ds(chunk * CHUNK, CHUNK)],
                        dstv[b], jsem[b]).wait()
                    pltpu.async_copy(rows[b], acc_sp.at[dstv[b]],
                                     ssem[b], add=True)

        for b in range(RBUF):
            @pl.when(b * NW + w < n_chunks)
            def _():
                pltpu.make_async_copy(rows[b], acc_sp.at[dstv[b]],
                                      ssem[b]).wait()

        plsc.subcore_barrier()
        pltpu.sync_copy(acc_sp.at[pl.ds(s * rows_per_sub, rows_per_sub)],
                        out_hbm.at[c, pl.ds(s * rows_per_sub, rows_per_sub)])

    return scatter_kernel(g, src_flat, dst_flat)


def _scale_tc(x, hist):
    n, d = x.shape
    bm = 2000

    def body(x_ref, hist_ref, u_ref, dis_ref):
        deg = hist_ref[0, :, 0:1] + hist_ref[1, :, 0:1] + 1.0
        dis = lax.rsqrt(deg)
        dis_ref[...] = dis
        u_ref[...] = x_ref[...] * dis

    return pl.pallas_call(
        body,
        grid=(n // bm,),
        in_specs=[
            pl.BlockSpec((bm, d), lambda i: (i, 0)),
            pl.BlockSpec((NC, bm, HW), lambda i: (0, i, 0)),
        ],
        out_specs=[
            pl.BlockSpec((bm, d), lambda i: (i, 0)),
            pl.BlockSpec((bm, 1), lambda i: (i, 0)),
        ],
        out_shape=[
            jax.ShapeDtypeStruct((n, d), jnp.float32),
            jax.ShapeDtypeStruct((n, 1), jnp.float32),
        ],
    )(x, hist)


def _combine_tc(partials, u, dis, W, b2):
    n, d = u.shape
    d_out = W.shape[1]
    bm = 2000

    def body(p_ref, u_ref, dis_ref, w_ref, b_ref, o_ref):
        t = p_ref[0] + p_ref[1] + u_ref[...]
        tw = lax.dot_general(
            t, w_ref[...], (((1,), (0,)), ((), ())),
            precision=lax.Precision.HIGHEST,
            preferred_element_type=jnp.float32)
        o_ref[...] = dis_ref[...] * tw + b_ref[...]

    return pl.pallas_call(
        body,
        grid=(n // bm,),
        in_specs=[
            pl.BlockSpec((NC, bm, d), lambda i: (0, i, 0)),
            pl.BlockSpec((bm, d), lambda i: (i, 0)),
            pl.BlockSpec((bm, 1), lambda i: (i, 0)),
            pl.BlockSpec((d, d_out), lambda i: (0, 0)),
            pl.BlockSpec((1, d_out), lambda i: (0, 0)),
        ],
        out_specs=pl.BlockSpec((bm, d_out), lambda i: (i, 0)),
        out_shape=jax.ShapeDtypeStruct((n, d_out), jnp.float32),
    )(partials, u, dis, W, b2)


def kernel(x, edge_index, W, b):
    ei = edge_index.astype(jnp.int32)
    dst2, n_chunks, per_w = _pad_chunks(ei[1])

    hist = _hist_sc(dst2, n_chunks, per_w, x.shape[0])
    u, dis = _scale_tc(x, hist)
    partials = _scatter_sc(u, ei[0], ei[1])
    return _combine_tc(partials, u, dis, W, b.reshape(1, -1))
